# baseline (speedup 1.0000x reference)
import jax
import jax.numpy as jnp
from jax.experimental import pallas as pl
from jax.experimental.pallas import tpu as pltpu

_N = 8192
_D = 32
_MARGIN = 0.5
_K = 256
_BLK = 256
_NBLK = _N // _BLK
_NEG = -3.0e38
_TINY = 1e-30


def _rowmax_kernel(inv_ref, a_ref, c_ref, a2_ref, b2_ref, ram_ref, rb_ref,
                   out_ref, buf_ref, sem_ref):
    i = pl.program_id(0)

    def _copy(blk, slot):
        return pltpu.make_async_copy(
            inv_ref.at[pl.ds(blk * _BLK, _BLK), :], buf_ref.at[slot],
            sem_ref.at[slot])

    @pl.when(i == 0)
    def _():
        _copy(0, 0).start()

    @pl.when(i + 1 < _NBLK)
    def _():
        _copy(i + 1, (i + 1) % 2).start()

    slot = jax.lax.rem(i, 2)
    _copy(i, slot).wait()
    inv = buf_ref[slot] != 0

    a = a_ref[...]
    ab = jax.lax.dot_general(
        a, c_ref[...], (((1,), (1,)), ((), ())),
        preferred_element_type=jnp.float32)
    d2 = (a2_ref[...] + b2_ref[...]) - 2.0 * ab
    c = jnp.maximum(d2, _TINY)
    dist = c * jax.lax.rsqrt(c)
    ov = (ram_ref[...] + rb_ref[...]) - dist
    m = jnp.max(jnp.where(inv, ov, _NEG), axis=1)
    out_ref[0, 0, :] = jnp.where(m > -1e30, jnp.maximum(m, 0.0), -1.0)


def _bisect(v, lo0, hi0, iters):
    kf = jnp.float32(_K)

    def body(_, carry):
        lo, hi = carry
        mid = 0.5 * (lo + hi)
        cnt = jnp.sum((v >= mid).astype(jnp.float32))
        ge = cnt >= kf
        return jnp.where(ge, mid, lo), jnp.where(ge, hi, mid)

    lo, _ = jax.lax.fori_loop(0, iters, body, (lo0, hi0))
    return lo


def _topsum_kernel(ridx_ref, inv_ref, ridxc_ref, c_ref, rad_ref, b2_ref,
                   rb_ref, out_ref, acc_ref, sem):
    cps = [
        pltpu.make_async_copy(inv_ref.at[ridx_ref[r]], acc_ref.at[r], sem)
        for r in range(_K)
    ]
    for cp in cps:
        cp.start()

    kf = jnp.float32(_K)
    lanes = jax.lax.broadcasted_iota(jnp.int32, (_K, _N), 1)
    oh = (lanes == ridxc_ref[...]).astype(jnp.float32)
    a_sel = jax.lax.dot_general(
        oh, c_ref[...], (((1,), (0,)), ((), ())),
        preferred_element_type=jnp.float32)
    r_sel = jax.lax.dot_general(
        oh, rad_ref[...], (((1,), (0,)), ((), ())),
        preferred_element_type=jnp.float32)
    ab = jax.lax.dot_general(
        a_sel, c_ref[...], (((1,), (1,)), ((), ())),
        preferred_element_type=jnp.float32)
    a2 = jnp.sum(a_sel * a_sel, axis=1, keepdims=True)
    d2 = (a2 + b2_ref[...]) - 2.0 * ab
    dist = jnp.sqrt(jnp.maximum(d2, 0.0))
    ov = (_MARGIN + r_sel + rb_ref[...]) - dist

    for cp in cps:
        cp.wait()
    inv = jnp.reshape(acc_ref[...].astype(jnp.float32), (_K, _N))
    v = jnp.where(inv > 0.5, jnp.maximum(ov, 0.0), -1.0)

    cm = jnp.max(v, axis=0, keepdims=True)
    tc = _bisect(cm, jnp.float32(-1.0), jnp.max(cm) + 1.0, 34)
    cmr = jnp.concatenate(
        [cm[:, c * 128:(c + 1) * 128] for c in range(_N // 128)],
        axis=0)
    gt = (cmr > tc).astype(jnp.float32)
    eq = (cmr == tc).astype(jnp.float32)
    nrow, nlan = cmr.shape
    li = jax.lax.broadcasted_iota(jnp.int32, (nlan, nlan), 0)
    lj = jax.lax.broadcasted_iota(jnp.int32, (nlan, nlan), 1)
    upper_incl = (li <= lj).astype(jnp.float32)
    ri = jax.lax.broadcasted_iota(jnp.int32, (nrow, nrow), 0)
    rj = jax.lax.broadcasted_iota(jnp.int32, (nrow, nrow), 1)
    low_strict = (rj < ri).astype(jnp.float32)
    ones_col = jnp.ones((nlan, 1), jnp.float32)

    def excl_rank(mask):
        prefix_in = jax.lax.dot_general(
            mask, upper_incl, (((1,), (0,)), ((), ())),
            preferred_element_type=jnp.float32)
        rows = jax.lax.dot_general(
            mask, ones_col, (((1,), (0,)), ((), ())),
            preferred_element_type=jnp.float32)
        off = jax.lax.dot_general(
            low_strict, rows, (((1,), (0,)), ((), ())),
            preferred_element_type=jnp.float32)
        return prefix_in - mask + off

    c_gt = jnp.sum(gt)
    slot = jnp.where(gt > 0.5, excl_rank(gt), c_gt + excl_rank(eq))
    valid = jnp.logical_and(jnp.logical_or(gt > 0.5, eq > 0.5), slot < kf)
    slot_i = jnp.where(valid, slot, -1.0).astype(jnp.int32)
    slot_row = jnp.concatenate(
        [slot_i[c:c + 1, :] for c in range(_N // 128)], axis=1)
    ks = jax.lax.broadcasted_iota(jnp.int32, (_K, _N), 0)
    selT = (ks == slot_row).astype(jnp.float32)
    vsub = jax.lax.dot_general(
        v, selT, (((1,), (1,)), ((), ())),
        preferred_element_type=jnp.float32)

    t = _bisect(vsub, tc, jnp.max(vsub) + 1.0, 34)
    gtv = vsub > t
    cnt_gt = jnp.sum(gtv.astype(jnp.float32))
    sum_gt = jnp.sum(jnp.where(gtv, vsub, 0.0))
    out_ref[...] = jnp.full((1, 1), sum_gt + (kf - cnt_gt) * t, jnp.float32)


@jax.jit
def _run(concept, radius, invalids):
    n = _N
    b2 = jnp.sum(concept * concept, axis=1)
    b2row = b2.reshape(1, n)
    rrow = radius.reshape(1, n)
    ramcol = (_MARGIN + radius).reshape(n, 1)

    rowmax3 = pl.pallas_call(
        _rowmax_kernel,
        grid=(_NBLK,),
        in_specs=[
            pl.BlockSpec(memory_space=pl.ANY),
            pl.BlockSpec((_BLK, _D), lambda i: (i, 0)),
            pl.BlockSpec((n, _D), lambda i: (0, 0)),
            pl.BlockSpec((_BLK, 1), lambda i: (i, 0)),
            pl.BlockSpec((1, n), lambda i: (0, 0)),
            pl.BlockSpec((_BLK, 1), lambda i: (i, 0)),
            pl.BlockSpec((1, n), lambda i: (0, 0)),
        ],
        out_specs=pl.BlockSpec((1, 1, _BLK), lambda i: (i, 0, 0)),
        out_shape=jax.ShapeDtypeStruct((_NBLK, 1, _BLK), jnp.float32),
        scratch_shapes=[
            pltpu.VMEM((2, _BLK, n), jnp.int8),
            pltpu.SemaphoreType.DMA((2,)),
        ],
    )(invalids.view(jnp.int8), concept, concept, b2.reshape(n, 1), b2row,
      ramcol, rrow)
    rowmax = rowmax3.reshape(n)

    _, ridx = jax.lax.top_k(rowmax, _K)
    ridx = ridx.astype(jnp.int32)

    total = pl.pallas_call(
        _topsum_kernel,
        grid_spec=pltpu.PrefetchScalarGridSpec(
            num_scalar_prefetch=1,
            grid=(1,),
            in_specs=[
                pl.BlockSpec(memory_space=pl.ANY),
                pl.BlockSpec((_K, 1), lambda i, r: (0, 0)),
                pl.BlockSpec((n, _D), lambda i, r: (0, 0)),
                pl.BlockSpec((n, 1), lambda i, r: (0, 0)),
                pl.BlockSpec((1, n), lambda i, r: (0, 0)),
                pl.BlockSpec((1, n), lambda i, r: (0, 0)),
            ],
            out_specs=pl.BlockSpec((1, 1), lambda i, r: (0, 0)),
            scratch_shapes=[
                pltpu.VMEM((_K, 64, 128), jnp.int8),
                pltpu.SemaphoreType.DMA,
            ],
        ),
        out_shape=jax.ShapeDtypeStruct((1, 1), jnp.float32),
    )(ridx,
      invalids.view(jnp.int8).reshape(n, 64, 128),
      ridx.reshape(_K, 1), concept, radius.reshape(n, 1), b2row, rrow)
    return total[0, 0]


def kernel(concept, radius, invalids, n_samples):
    return _run(concept, radius, invalids) / n_samples


# speedup vs baseline: 1.0043x; 1.0043x over previous
import jax
import jax.numpy as jnp
from jax.experimental import pallas as pl
from jax.experimental.pallas import tpu as pltpu

_N = 8192
_D = 32
_MARGIN = 0.5
_K = 256
_BLK = 256
_NBLK = _N // _BLK
_NEG = -3.0e38
_TINY = 1e-30


def _rowmax_kernel(inv_ref, a_ref, c_ref, a2_ref, b2_ref, ram_ref, rb_ref,
                   out_ref, buf_ref, sem_ref):
    i = pl.program_id(0)

    def _copy(blk, slot):
        return pltpu.make_async_copy(
            inv_ref.at[pl.ds(blk * _BLK, _BLK), :], buf_ref.at[slot],
            sem_ref.at[slot])

    @pl.when(i == 0)
    def _():
        _copy(0, 0).start()
        _copy(1, 1).start()

    @pl.when(i + 2 < _NBLK)
    def _():
        _copy(i + 2, (i + 2) % 3).start()

    slot = jax.lax.rem(i, 3)
    _copy(i, slot).wait()
    inv = buf_ref[slot] != 0

    a = a_ref[...]
    ab = jax.lax.dot_general(
        a, c_ref[...], (((1,), (1,)), ((), ())),
        preferred_element_type=jnp.float32)
    d2 = (a2_ref[...] + b2_ref[...]) - 2.0 * ab
    c = jnp.maximum(d2, _TINY)
    dist = c * jax.lax.rsqrt(c)
    ov = (ram_ref[...] + rb_ref[...]) - dist
    m = jnp.max(jnp.where(inv, ov, _NEG), axis=1)
    out_ref[0, 0, :] = jnp.where(m > -1e30, jnp.maximum(m, 0.0), -1.0)


def _bisect(v, lo0, hi0, iters):
    kf = jnp.float32(_K)

    def body(_, carry):
        lo, hi = carry
        mid = 0.5 * (lo + hi)
        cnt = jnp.sum((v >= mid).astype(jnp.float32))
        ge = cnt >= kf
        return jnp.where(ge, mid, lo), jnp.where(ge, hi, mid)

    lo, _ = jax.lax.fori_loop(0, iters, body, (lo0, hi0))
    return lo


def _topsum_kernel(ridx_ref, inv_ref, ridxc_ref, c_ref, rad_ref, b2_ref,
                   rb_ref, out_ref, acc_ref, sem):
    cps = [
        pltpu.make_async_copy(inv_ref.at[ridx_ref[r]], acc_ref.at[r], sem)
        for r in range(_K)
    ]
    for cp in cps:
        cp.start()

    kf = jnp.float32(_K)
    lanes = jax.lax.broadcasted_iota(jnp.int32, (_K, _N), 1)
    oh = (lanes == ridxc_ref[...]).astype(jnp.float32)
    a_sel = jax.lax.dot_general(
        oh, c_ref[...], (((1,), (0,)), ((), ())),
        preferred_element_type=jnp.float32)
    r_sel = jax.lax.dot_general(
        oh, rad_ref[...], (((1,), (0,)), ((), ())),
        preferred_element_type=jnp.float32)
    ab = jax.lax.dot_general(
        a_sel, c_ref[...], (((1,), (1,)), ((), ())),
        preferred_element_type=jnp.float32)
    a2 = jnp.sum(a_sel * a_sel, axis=1, keepdims=True)
    d2 = (a2 + b2_ref[...]) - 2.0 * ab
    dist = jnp.sqrt(jnp.maximum(d2, 0.0))
    ov = (_MARGIN + r_sel + rb_ref[...]) - dist

    for cp in cps:
        cp.wait()
    inv = jnp.reshape(acc_ref[...].astype(jnp.float32), (_K, _N))
    v = jnp.where(inv > 0.5, jnp.maximum(ov, 0.0), -1.0)

    cm = jnp.max(v, axis=0, keepdims=True)
    tc = _bisect(cm, jnp.float32(-1.0), jnp.max(cm) + 1.0, 34)
    cmr = jnp.concatenate(
        [cm[:, c * 128:(c + 1) * 128] for c in range(_N // 128)],
        axis=0)
    gt = (cmr > tc).astype(jnp.float32)
    eq = (cmr == tc).astype(jnp.float32)
    nrow, nlan = cmr.shape
    li = jax.lax.broadcasted_iota(jnp.int32, (nlan, nlan), 0)
    lj = jax.lax.broadcasted_iota(jnp.int32, (nlan, nlan), 1)
    upper_incl = (li <= lj).astype(jnp.float32)
    ri = jax.lax.broadcasted_iota(jnp.int32, (nrow, nrow), 0)
    rj = jax.lax.broadcasted_iota(jnp.int32, (nrow, nrow), 1)
    low_strict = (rj < ri).astype(jnp.float32)
    ones_col = jnp.ones((nlan, 1), jnp.float32)

    def excl_rank(mask):
        prefix_in = jax.lax.dot_general(
            mask, upper_incl, (((1,), (0,)), ((), ())),
            preferred_element_type=jnp.float32)
        rows = jax.lax.dot_general(
            mask, ones_col, (((1,), (0,)), ((), ())),
            preferred_element_type=jnp.float32)
        off = jax.lax.dot_general(
            low_strict, rows, (((1,), (0,)), ((), ())),
            preferred_element_type=jnp.float32)
        return prefix_in - mask + off

    c_gt = jnp.sum(gt)
    slot = jnp.where(gt > 0.5, excl_rank(gt), c_gt + excl_rank(eq))
    valid = jnp.logical_and(jnp.logical_or(gt > 0.5, eq > 0.5), slot < kf)
    slot_i = jnp.where(valid, slot, -1.0).astype(jnp.int32)
    slot_row = jnp.concatenate(
        [slot_i[c:c + 1, :] for c in range(_N // 128)], axis=1)
    ks = jax.lax.broadcasted_iota(jnp.int32, (_K, _N), 0)
    selT = (ks == slot_row).astype(jnp.float32)
    vsub = jax.lax.dot_general(
        v, selT, (((1,), (1,)), ((), ())),
        preferred_element_type=jnp.float32)

    t = _bisect(vsub, tc, jnp.max(vsub) + 1.0, 34)
    gtv = vsub > t
    cnt_gt = jnp.sum(gtv.astype(jnp.float32))
    sum_gt = jnp.sum(jnp.where(gtv, vsub, 0.0))
    out_ref[...] = jnp.full((1, 1), sum_gt + (kf - cnt_gt) * t, jnp.float32)


@jax.jit
def _run(concept, radius, invalids):
    n = _N
    b2 = jnp.sum(concept * concept, axis=1)
    b2row = b2.reshape(1, n)
    rrow = radius.reshape(1, n)
    ramcol = (_MARGIN + radius).reshape(n, 1)

    rowmax3 = pl.pallas_call(
        _rowmax_kernel,
        grid=(_NBLK,),
        in_specs=[
            pl.BlockSpec(memory_space=pl.ANY),
            pl.BlockSpec((_BLK, _D), lambda i: (i, 0)),
            pl.BlockSpec((n, _D), lambda i: (0, 0)),
            pl.BlockSpec((_BLK, 1), lambda i: (i, 0)),
            pl.BlockSpec((1, n), lambda i: (0, 0)),
            pl.BlockSpec((_BLK, 1), lambda i: (i, 0)),
            pl.BlockSpec((1, n), lambda i: (0, 0)),
        ],
        out_specs=pl.BlockSpec((1, 1, _BLK), lambda i: (i, 0, 0)),
        out_shape=jax.ShapeDtypeStruct((_NBLK, 1, _BLK), jnp.float32),
        scratch_shapes=[
            pltpu.VMEM((3, _BLK, n), jnp.int8),
            pltpu.SemaphoreType.DMA((3,)),
        ],
    )(invalids.view(jnp.int8), concept, concept, b2.reshape(n, 1), b2row,
      ramcol, rrow)
    rowmax = rowmax3.reshape(n)

    _, ridx = jax.lax.top_k(rowmax, _K)
    ridx = ridx.astype(jnp.int32)

    total = pl.pallas_call(
        _topsum_kernel,
        grid_spec=pltpu.PrefetchScalarGridSpec(
            num_scalar_prefetch=1,
            grid=(1,),
            in_specs=[
                pl.BlockSpec(memory_space=pl.ANY),
                pl.BlockSpec((_K, 1), lambda i, r: (0, 0)),
                pl.BlockSpec((n, _D), lambda i, r: (0, 0)),
                pl.BlockSpec((n, 1), lambda i, r: (0, 0)),
                pl.BlockSpec((1, n), lambda i, r: (0, 0)),
                pl.BlockSpec((1, n), lambda i, r: (0, 0)),
            ],
            out_specs=pl.BlockSpec((1, 1), lambda i, r: (0, 0)),
            scratch_shapes=[
                pltpu.VMEM((_K, 64, 128), jnp.int8),
                pltpu.SemaphoreType.DMA,
            ],
        ),
        out_shape=jax.ShapeDtypeStruct((1, 1), jnp.float32),
    )(ridx,
      invalids.view(jnp.int8).reshape(n, 64, 128),
      ridx.reshape(_K, 1), concept, radius.reshape(n, 1), b2row, rrow)
    return total[0, 0]


def kernel(concept, radius, invalids, n_samples):
    return _run(concept, radius, invalids) / n_samples
